# baseline (speedup 1.0000x reference)
import functools
import math

import jax
import jax.numpy as jnp
from jax import lax
from jax.experimental import pallas as pl
from jax.experimental.pallas import tpu as pltpu
from jax.experimental.pallas import tpu_sc as plsc

_N = 2048
_V = 100000
_PAD = 0
_SMOOTH = 0.1
_CONF = 1.0 - _SMOOTH
_SV = _SMOOTH / (_V - 2)
_C0 = (_V - 2) * _SV * math.log(_SV) + _CONF * math.log(_CONF)
_NEG = float("-inf")
_BIG = 2 ** 30

_NT = 1024
_NS_ROWS = _N - _NT

_BR = 256
_BC = 8192
_G = _BC // 128
_RS = 16
_NSTRIP = _BR // _RS
_RB = _NT // _BR
_CB = -(-_V // _BC)
_LAST_BASE = (_CB - 1) * _BC
_REM = _V - _LAST_BASE
_NG_LAST = -(-_REM // 128)
_MASK_G = _NG_LAST - 1 if _REM % 128 else -1


def _tc_body(x_ref, t_ref, xt_ref, x0_ref, loss_ref, cor_ref, np_ref,
             s_ref, m_ref, i_ref):
    r = pl.program_id(0)
    c = pl.program_id(1)

    @pl.when(c == 0)
    def _():
        s_ref[...] = jnp.zeros((_BR, 128), jnp.float32)
        m_ref[...] = jnp.full((_BR, 128), _NEG, jnp.float32)
        i_ref[...] = jnp.zeros((_BR, 128), jnp.int32)

    def _update(ngroups, mask_group):
        for sidx in range(_NSTRIP):
            rows = slice(sidx * _RS, (sidx + 1) * _RS)
            s = s_ref[rows, :]
            m = m_ref[rows, :]
            i = i_ref[rows, :]
            for k in range(ngroups):
                xg = x_ref[rows, k * 128:(k + 1) * 128]
                if k == mask_group:
                    lane = lax.broadcasted_iota(jnp.int32, (_RS, 128), 1)
                    valid = lane < (_REM - k * 128)
                    xs = jnp.where(valid, xg, 0.0)
                    xm = jnp.where(valid, xg, _NEG)
                else:
                    xs = xg
                    xm = xg
                s = s + xs
                upd = xm > m
                m = jnp.maximum(m, xm)
                i = jnp.where(upd, c * _G + k, i)
            s_ref[rows, :] = s
            m_ref[rows, :] = m
            i_ref[rows, :] = i

    @pl.when(c < _CB - 1)
    def _():
        _update(_G, -1)

    @pl.when(c == _CB - 1)
    def _():
        _update(_NG_LAST, _MASK_G)
        s = s_ref[...]
        m = m_ref[...]
        i = i_ref[...]
        lane = lax.broadcasted_iota(jnp.int32, (_BR, 128), 1)
        col = i * 128 + lane
        rsum = jnp.sum(s, axis=1, keepdims=True)
        rmax = jnp.max(m, axis=1, keepdims=True)
        first = jnp.min(jnp.where(m == rmax, col, _BIG), axis=1,
                        keepdims=True)
        t = t_ref[...]
        xt = xt_ref[...]
        x0 = x0_ref[...]
        nonpad = t != _PAD
        lrows = jnp.where(nonpad,
                          _C0 - _SV * (rsum - x0 - xt) - _CONF * xt, 0.0)
        part_loss = jnp.sum(lrows)
        part_cor = jnp.sum(jnp.where(nonpad & (first == t), 1, 0))
        part_np = jnp.sum(nonpad.astype(jnp.int32))

        @pl.when(r == 0)
        def _():
            loss_ref[0, 0] = part_loss
            cor_ref[0, 0] = part_cor
            np_ref[0, 0] = part_np

        @pl.when(r > 0)
        def _():
            loss_ref[0, 0] = loss_ref[0, 0] + part_loss
            cor_ref[0, 0] = cor_ref[0, 0] + part_cor
            np_ref[0, 0] = np_ref[0, 0] + part_np


def _tc_main(xa, t2, xt2, x02):
    return pl.pallas_call(
        _tc_body,
        grid=(_RB, _CB),
        in_specs=[
            pl.BlockSpec((_BR, _BC), lambda r, c: (r, c)),
            pl.BlockSpec((_BR, 1), lambda r, c: (r, 0)),
            pl.BlockSpec((_BR, 1), lambda r, c: (r, 0)),
            pl.BlockSpec((_BR, 1), lambda r, c: (r, 0)),
        ],
        out_specs=[
            pl.BlockSpec(memory_space=pltpu.SMEM),
            pl.BlockSpec(memory_space=pltpu.SMEM),
            pl.BlockSpec(memory_space=pltpu.SMEM),
        ],
        out_shape=[
            jax.ShapeDtypeStruct((1, 1), jnp.float32),
            jax.ShapeDtypeStruct((1, 1), jnp.int32),
            jax.ShapeDtypeStruct((1, 1), jnp.int32),
        ],
        scratch_shapes=[
            pltpu.VMEM((_BR, 128), jnp.float32),
            pltpu.VMEM((_BR, 128), jnp.float32),
            pltpu.VMEM((_BR, 128), jnp.int32),
        ],
    )(xa, t2, xt2, x02)


_NW = 32
_RPW = _NS_ROWS // _NW
_GPW = _RPW // 8
_NCH = 32
_CW = 3200
_LASTW = 800
_LASTV = _LASTW // 16


def _group_chunk(buf, states, c32, nv):
    vb = c32 * (_CW // 16)

    def body(jj, st):
        out = []
        vecidx = vb + jj
        for r8 in range(8):
            acc, m, i = st[r8]
            v = buf[pl.ds(r8 * _CW + jj * 16, 16)]
            acc = acc + v
            upd = v > m
            m = jnp.maximum(m, v)
            i = jnp.where(upd, vecidx, i)
            out.append((acc, m, i))
        return tuple(out)

    return lax.fori_loop(0, nv, body, states)


def _sc_main_body(x_hbm, s_hbm, m_hbm, i_hbm,
                  buf0, buf1, s_st, m_st, i_st, sem0, sem1):
    wid = lax.axis_index("s") * 2 + lax.axis_index("c")
    r0 = _NT + wid * _RPW
    bufs = (buf0, buf1)
    sems = (sem0, sem1)

    def issue(grp, c32, b):
        w = _CW if c32 < _NCH - 1 else _LASTW
        for r8 in range(8):
            row = r0 + grp * 8 + r8
            pltpu.async_copy(
                x_hbm.at[pl.ds(row * _V + c32 * _CW, w)],
                bufs[b].at[pl.ds(r8 * _CW, w)], sems[b])

    def wait(c32, b):
        w = _CW if c32 < _NCH - 1 else _LASTW
        for r8 in range(8):
            pltpu.make_async_copy(
                x_hbm.at[pl.ds(0, w)],
                bufs[b].at[pl.ds(r8 * _CW, w)], sems[b]).wait()

    issue(0, 0, 0)

    def group_body(g, _):
        st = tuple((jnp.zeros((16,), jnp.float32),
                    jnp.full((16,), _NEG, jnp.float32),
                    jnp.zeros((16,), jnp.int32)) for _ in range(8))
        for c32 in range(_NCH):
            b = c32 % 2
            wait(c32, b)
            if c32 < _NCH - 1:
                issue(g, c32 + 1, 1 - b)
            else:
                issue(jnp.minimum(g + 1, _GPW - 1), 0, 1 - b)
            nv = _CW // 16 if c32 < _NCH - 1 else _LASTV
            st = _group_chunk(bufs[b], st, c32, nv)
        for r8 in range(8):
            acc, m, i = st[r8]
            base = (g * 8 + r8) * 16
            s_st[pl.ds(base, 16)] = acc
            m_st[pl.ds(base, 16)] = m
            i_st[pl.ds(base, 16)] = i
        return 0

    lax.fori_loop(0, _GPW, group_body, 0)
    wait(0, 0)
    out0 = wid * _RPW * 16
    pltpu.sync_copy(s_st, s_hbm.at[pl.ds(out0, _RPW * 16)])
    pltpu.sync_copy(m_st, m_hbm.at[pl.ds(out0, _RPW * 16)])
    pltpu.sync_copy(i_st, i_hbm.at[pl.ds(out0, _RPW * 16)])


@functools.cache
def _sc_main():
    return pl.kernel(
        _sc_main_body,
        out_type=[jax.ShapeDtypeStruct((_NS_ROWS * 16,), jnp.float32),
                  jax.ShapeDtypeStruct((_NS_ROWS * 16,), jnp.float32),
                  jax.ShapeDtypeStruct((_NS_ROWS * 16,), jnp.int32)],
        mesh=plsc.VectorSubcoreMesh(core_axis_name="c",
                                    subcore_axis_name="s"),
        scratch_types=[
            pltpu.VMEM((8 * _CW,), jnp.float32),
            pltpu.VMEM((8 * _CW,), jnp.float32),
            pltpu.VMEM((_RPW * 16,), jnp.float32),
            pltpu.VMEM((_RPW * 16,), jnp.float32),
            pltpu.VMEM((_RPW * 16,), jnp.int32),
            pltpu.SemaphoreType.DMA,
            pltpu.SemaphoreType.DMA,
        ],
    )


_RPWG = _N // _NW
_CHUNKS = _RPWG // 16


def _sc_gather_body(flat_hbm, tgt_hbm, xt_hbm, x0_hbm,
                    tgt_v, idxt_v, idx0_v, xt_v, x0_v, sem):
    wid = lax.axis_index("s") * 2 + lax.axis_index("c")
    base = wid * _RPWG
    pltpu.sync_copy(tgt_hbm.at[pl.ds(base, _RPWG)], tgt_v)
    iota = lax.iota(jnp.int32, 16)
    for k in range(_CHUNKS):
        rows = iota + (base + k * 16)
        t16 = tgt_v[pl.ds(k * 16, 16)]
        idx0_v[pl.ds(k * 16, 16)] = rows * _V
        idxt_v[pl.ds(k * 16, 16)] = rows * _V + t16
    pltpu.async_copy(flat_hbm.at[idxt_v], xt_v, sem).wait()
    pltpu.async_copy(flat_hbm.at[idx0_v], x0_v, sem).wait()
    pltpu.sync_copy(xt_v, xt_hbm.at[pl.ds(base, _RPWG)])
    pltpu.sync_copy(x0_v, x0_hbm.at[pl.ds(base, _RPWG)])


@functools.cache
def _sc_gather():
    return pl.kernel(
        _sc_gather_body,
        out_type=[jax.ShapeDtypeStruct((_N,), jnp.float32),
                  jax.ShapeDtypeStruct((_N,), jnp.float32)],
        mesh=plsc.VectorSubcoreMesh(core_axis_name="c",
                                    subcore_axis_name="s"),
        scratch_types=[
            pltpu.VMEM((_RPWG,), jnp.int32),
            pltpu.VMEM((_RPWG,), jnp.int32),
            pltpu.VMEM((_RPWG,), jnp.int32),
            pltpu.VMEM((_RPWG,), jnp.float32),
            pltpu.VMEM((_RPWG,), jnp.float32),
            pltpu.SemaphoreType.DMA,
        ],
    )


def _fin_body(s_ref, m_ref, i_ref, t_ref, xt_ref, x0_ref,
              loss_ref, cor_ref, np_ref):
    s16 = s_ref[...]
    m16 = m_ref[...]
    i16 = i_ref[...]
    lane = lax.broadcasted_iota(jnp.int32, (_NS_ROWS, 16), 1)
    rsum = jnp.sum(s16, axis=1, keepdims=True)
    rmax = jnp.max(m16, axis=1, keepdims=True)
    cols = i16 * 16 + lane
    first = jnp.min(jnp.where(m16 == rmax, cols, _BIG), axis=1,
                    keepdims=True)
    t = t_ref[...]
    xt = xt_ref[...]
    x0 = x0_ref[...]
    nonpad = t != _PAD
    lrows = jnp.where(nonpad,
                      _C0 - _SV * (rsum - x0 - xt) - _CONF * xt, 0.0)
    loss_ref[0, 0] = jnp.sum(lrows)
    cor_ref[0, 0] = jnp.sum(jnp.where(nonpad & (first == t), 1, 0))
    np_ref[0, 0] = jnp.sum(nonpad.astype(jnp.int32))


def _finalize(s, m, i, t, xt, x0):
    return pl.pallas_call(
        _fin_body,
        out_specs=[
            pl.BlockSpec(memory_space=pltpu.SMEM),
            pl.BlockSpec(memory_space=pltpu.SMEM),
            pl.BlockSpec(memory_space=pltpu.SMEM),
        ],
        out_shape=[
            jax.ShapeDtypeStruct((1, 1), jnp.float32),
            jax.ShapeDtypeStruct((1, 1), jnp.int32),
            jax.ShapeDtypeStruct((1, 1), jnp.int32),
        ],
    )(s.reshape(_NS_ROWS, 16), m.reshape(_NS_ROWS, 16),
      i.reshape(_NS_ROWS, 16), t.reshape(_NS_ROWS, 1),
      xt.reshape(_NS_ROWS, 1), x0.reshape(_NS_ROWS, 1))


def kernel(output, target):
    target = target.astype(jnp.int32)
    flat = output.reshape(_N * _V)
    xt, x0 = _sc_gather()(flat, target)
    s, m, i = _sc_main()(flat)
    la, ca, na = _tc_main(output[:_NT], target[:_NT].reshape(_NT, 1),
                          xt[:_NT].reshape(_NT, 1), x0[:_NT].reshape(_NT, 1))
    lb, cb, nb = _finalize(s, m, i, target[_NT:], xt[_NT:], x0[_NT:])
    return (la[0, 0] + lb[0, 0], ca[0, 0] + cb[0, 0], na[0, 0] + nb[0, 0])


# speedup vs baseline: 1.0769x; 1.0769x over previous
import functools
import math

import jax
import jax.numpy as jnp
from jax import lax
from jax.experimental import pallas as pl
from jax.experimental.pallas import tpu as pltpu
from jax.experimental.pallas import tpu_sc as plsc

_N = 2048
_V = 100000
_PAD = 0
_SMOOTH = 0.1
_CONF = 1.0 - _SMOOTH
_SV = _SMOOTH / (_V - 2)
_C0 = (_V - 2) * _SV * math.log(_SV) + _CONF * math.log(_CONF)

_BR = 256
_BC = 8192
_G = _BC // 128
_RS = 16
_NS = _BR // _RS
_RB = _N // _BR
_CB = -(-_V // _BC)
_LAST_BASE = (_CB - 1) * _BC
_REM = _V - _LAST_BASE
_NG_LAST = -(-_REM // 128)
_MASK_G = _NG_LAST - 1 if _REM % 128 else -1
_NEG = float("-inf")
_BIG = 2 ** 30


def _tc_body(x_ref, t_ref, xt_ref, x0_ref, loss_ref, cor_ref, np_ref,
             s_ref, m_ref, i_ref):
    r = pl.program_id(0)
    c = pl.program_id(1)

    @pl.when(c == 0)
    def _():
        s_ref[...] = jnp.zeros((_BR, 128), jnp.float32)
        m_ref[...] = jnp.full((_BR, 128), _NEG, jnp.float32)
        i_ref[...] = jnp.zeros((_BR, 128), jnp.int32)

    def _update(ngroups, mask_group):
        for sidx in range(_NS):
            rows = slice(sidx * _RS, (sidx + 1) * _RS)
            s = s_ref[rows, :]
            m = m_ref[rows, :]
            i = i_ref[rows, :]
            for k in range(ngroups):
                xg = x_ref[rows, k * 128:(k + 1) * 128]
                if k == mask_group:
                    lane = lax.broadcasted_iota(jnp.int32, (_RS, 128), 1)
                    valid = lane < (_REM - k * 128)
                    xs = jnp.where(valid, xg, 0.0)
                    xm = jnp.where(valid, xg, _NEG)
                else:
                    xs = xg
                    xm = xg
                s = s + xs
                upd = xm > m
                m = jnp.maximum(m, xm)
                i = jnp.where(upd, c * _G + k, i)
            s_ref[rows, :] = s
            m_ref[rows, :] = m
            i_ref[rows, :] = i

    @pl.when(c < _CB - 1)
    def _():
        _update(_G, -1)

    @pl.when(c == _CB - 1)
    def _():
        _update(_NG_LAST, _MASK_G)
        s = s_ref[...]
        m = m_ref[...]
        i = i_ref[...]
        lane = lax.broadcasted_iota(jnp.int32, (_BR, 128), 1)
        col = i * 128 + lane
        rsum = jnp.sum(s, axis=1, keepdims=True)
        rmax = jnp.max(m, axis=1, keepdims=True)
        first = jnp.min(jnp.where(m == rmax, col, _BIG), axis=1,
                        keepdims=True)
        t = t_ref[...]
        xt = xt_ref[...]
        x0 = x0_ref[...]
        nonpad = t != _PAD
        lrows = jnp.where(nonpad,
                          _C0 - _SV * (rsum - x0 - xt) - _CONF * xt, 0.0)
        part_loss = jnp.sum(lrows)
        part_cor = jnp.sum(jnp.where(nonpad & (first == t), 1, 0))
        part_np = jnp.sum(nonpad.astype(jnp.int32))

        @pl.when(r == 0)
        def _():
            loss_ref[0, 0] = part_loss
            cor_ref[0, 0] = part_cor
            np_ref[0, 0] = part_np

        @pl.when(r > 0)
        def _():
            loss_ref[0, 0] = loss_ref[0, 0] + part_loss
            cor_ref[0, 0] = cor_ref[0, 0] + part_cor
            np_ref[0, 0] = np_ref[0, 0] + part_np


def _tc_main(output, t2, xt2, x02, interpret=False):
    return pl.pallas_call(
        _tc_body,
        grid=(_RB, _CB),
        in_specs=[
            pl.BlockSpec((_BR, _BC), lambda r, c: (r, c)),
            pl.BlockSpec((_BR, 1), lambda r, c: (r, 0)),
            pl.BlockSpec((_BR, 1), lambda r, c: (r, 0)),
            pl.BlockSpec((_BR, 1), lambda r, c: (r, 0)),
        ],
        out_specs=[
            pl.BlockSpec(memory_space=pltpu.SMEM),
            pl.BlockSpec(memory_space=pltpu.SMEM),
            pl.BlockSpec(memory_space=pltpu.SMEM),
        ],
        out_shape=[
            jax.ShapeDtypeStruct((1, 1), jnp.float32),
            jax.ShapeDtypeStruct((1, 1), jnp.int32),
            jax.ShapeDtypeStruct((1, 1), jnp.int32),
        ],
        scratch_shapes=[
            pltpu.VMEM((_BR, 128), jnp.float32),
            pltpu.VMEM((_BR, 128), jnp.float32),
            pltpu.VMEM((_BR, 128), jnp.int32),
        ],
        interpret=interpret,
    )(output, t2, xt2, x02)


_NW = 32
_RPW = _N // _NW
_CHUNKS = _RPW // 16


def _sc_body(flat_hbm, tgt_hbm, xt_hbm, x0_hbm,
             tgt_v, idxt_v, idx0_v, xt_v, x0_v, sem):
    wid = lax.axis_index("s") * 2 + lax.axis_index("c")
    base = wid * _RPW
    pltpu.sync_copy(tgt_hbm.at[pl.ds(base, _RPW)], tgt_v)
    iota = lax.iota(jnp.int32, 16)
    for k in range(_CHUNKS):
        rows = iota + (base + k * 16)
        t16 = tgt_v[pl.ds(k * 16, 16)]
        idx0_v[pl.ds(k * 16, 16)] = rows * _V
        idxt_v[pl.ds(k * 16, 16)] = rows * _V + t16
    pltpu.async_copy(flat_hbm.at[idxt_v], xt_v, sem).wait()
    pltpu.async_copy(flat_hbm.at[idx0_v], x0_v, sem).wait()
    pltpu.sync_copy(xt_v, xt_hbm.at[pl.ds(base, _RPW)])
    pltpu.sync_copy(x0_v, x0_hbm.at[pl.ds(base, _RPW)])


@functools.cache
def _sc_gather():
    return pl.kernel(
        _sc_body,
        out_type=[jax.ShapeDtypeStruct((_N,), jnp.float32),
                  jax.ShapeDtypeStruct((_N,), jnp.float32)],
        mesh=plsc.VectorSubcoreMesh(core_axis_name="c",
                                    subcore_axis_name="s"),
        scratch_types=[
            pltpu.VMEM((_RPW,), jnp.int32),
            pltpu.VMEM((_RPW,), jnp.int32),
            pltpu.VMEM((_RPW,), jnp.int32),
            pltpu.VMEM((_RPW,), jnp.float32),
            pltpu.VMEM((_RPW,), jnp.float32),
            pltpu.SemaphoreType.DMA,
        ],
    )


def kernel(output, target):
    target = target.astype(jnp.int32)
    xt, x0 = _sc_gather()(output.reshape(_N * _V), target)
    loss, cor, npd = _tc_main(output, target.reshape(_N, 1),
                              xt.reshape(_N, 1), x0.reshape(_N, 1))
    return loss[0, 0], cor[0, 0], npd[0, 0]
